# Initial kernel scaffold; baseline (speedup 1.0000x reference)
import jax
import jax.numpy as jnp
from jax.experimental import pallas as pl


def kernel(features, edge_index, W_self1, W_neigh1, b1, W_self2, W_neigh2, b2, W_out, b_out):
    raise NotImplementedError("write your pallas kernel here")


# speedup vs baseline: 6.0729x; 6.0729x over previous
import functools

import jax
import jax.numpy as jnp
from jax import lax
from jax.experimental import pallas as pl
from jax.experimental.pallas import tpu as pltpu
from jax.experimental.pallas import tpu_sc as plsc

N_NODES = 10000
D = 128
N_CLASSES = 40

NC = 2
NS = 16
NW = NC * NS
K = 128
NE_PAD = 327680
EPW = NE_PAD // NW
NCH = EPW // K
ACC_ROWS = 10240
ZPT = ACC_ROWS // NS
RPT = 624


def _sc_agg_body(with_deg, table, src, dst, z2d, z1d, *rest):
    if with_deg:
        out, degout, acc, deg_s, srcbuf, dstbuf, rows, ones_v, degv, sem = rest
    else:
        out, acc, srcbuf, dstbuf, rows, sem = rest
    c = lax.axis_index("c")
    s = lax.axis_index("s")

    pltpu.sync_copy(z2d, acc.at[pl.ds(s * ZPT, ZPT)])
    if with_deg:
        pltpu.sync_copy(z1d, deg_s.at[pl.ds(s * ZPT, ZPT)])
        for j in range(K // 16):
            ones_v[pl.ds(j * 16, 16)] = jnp.ones((16,), jnp.float32)
    plsc.subcore_barrier()

    ebase = (c * NS + s) * EPW

    def chunk(i, carry):
        off = ebase + i * K
        pltpu.sync_copy(src.at[pl.ds(off, K)], srcbuf.at[0])
        pltpu.sync_copy(dst.at[pl.ds(off, K)], dstbuf.at[0])
        pltpu.async_copy(table.at[srcbuf.at[0]], rows.at[0], sem).wait()
        pltpu.sync_copy(rows.at[0], acc.at[dstbuf.at[0]], add=True)
        if with_deg:
            pltpu.sync_copy(ones_v, deg_s.at[dstbuf.at[0]], add=True)
        return carry

    lax.fori_loop(0, NCH, chunk, 0)
    plsc.subcore_barrier()

    pltpu.sync_copy(acc.at[pl.ds(s * RPT, RPT)], out.at[c, pl.ds(s * RPT, RPT)])

    @pl.when(s == 0)
    def _():
        pltpu.sync_copy(acc.at[pl.ds(NS * RPT, N_NODES - NS * RPT)],
                        out.at[c, pl.ds(NS * RPT, N_NODES - NS * RPT)])
        if with_deg:
            pltpu.sync_copy(deg_s.at[pl.ds(0, N_NODES)], degv)
            pltpu.sync_copy(degv, degout.at[pl.ds(c * N_NODES, N_NODES)])


def _make_sc_agg(with_deg):
    mesh = plsc.VectorSubcoreMesh(core_axis_name="c", subcore_axis_name="s")
    if with_deg:
        out_type = (
            jax.ShapeDtypeStruct((NC, N_NODES, D), jnp.float32),
            jax.ShapeDtypeStruct((NC * N_NODES,), jnp.float32),
        )
        scratch = [
            pltpu.VMEM_SHARED((ACC_ROWS, D), jnp.float32),
            pltpu.VMEM_SHARED((ACC_ROWS,), jnp.float32),
            pltpu.VMEM((1, K), jnp.int32),
            pltpu.VMEM((1, K), jnp.int32),
            pltpu.VMEM((1, K, D), jnp.float32),
            pltpu.VMEM((K,), jnp.float32),
            pltpu.VMEM((N_NODES,), jnp.float32),
            pltpu.SemaphoreType.DMA,
        ]
    else:
        out_type = jax.ShapeDtypeStruct((NC, N_NODES, D), jnp.float32)
        scratch = [
            pltpu.VMEM_SHARED((ACC_ROWS, D), jnp.float32),
            pltpu.VMEM((1, K), jnp.int32),
            pltpu.VMEM((1, K), jnp.int32),
            pltpu.VMEM((1, K, D), jnp.float32),
            pltpu.SemaphoreType.DMA,
        ]
    return pl.kernel(
        functools.partial(_sc_agg_body, with_deg),
        out_type=out_type,
        mesh=mesh,
        scratch_types=scratch,
    )


BR = 1000


def _mean_agg(sp_ref, degT_ref):
    ssum = sp_ref[0] + sp_ref[1]
    deg = jnp.sum(degT_ref[...], axis=1, keepdims=True)
    invd = 1.0 / jnp.clip(deg, 1.0, None)
    return ssum * invd


def _dense1_body(x_ref, sp_ref, degT_ref, Ws_ref, Wn_ref, b_ref, o_ref):
    agg = _mean_agg(sp_ref, degT_ref)
    h = jnp.dot(x_ref[...], Ws_ref[...], preferred_element_type=jnp.float32)
    h = h + jnp.dot(agg, Wn_ref[...], preferred_element_type=jnp.float32)
    h = h + b_ref[...][None, :]
    o_ref[...] = jnp.maximum(h, 0.0)


def _dense2_body(x_ref, sp_ref, degT_ref, Ws_ref, Wn_ref, b_ref, Wo_ref,
                 bo_ref, o_ref):
    agg = _mean_agg(sp_ref, degT_ref)
    h = jnp.dot(x_ref[...], Ws_ref[...], preferred_element_type=jnp.float32)
    h = h + jnp.dot(agg, Wn_ref[...], preferred_element_type=jnp.float32)
    h = jnp.maximum(h + b_ref[...][None, :], 0.0)
    o_ref[...] = (
        jnp.dot(h, Wo_ref[...], preferred_element_type=jnp.float32)
        + bo_ref[...][None, :]
    )


_W_SPEC = pl.BlockSpec((D, D), lambda i: (0, 0))
_B_SPEC = pl.BlockSpec((D,), lambda i: (0,))
_X_SPEC = pl.BlockSpec((BR, D), lambda i: (i, 0))
_SP_SPEC = pl.BlockSpec((NC, BR, D), lambda i: (0, i, 0))
_DEG_SPEC = pl.BlockSpec((BR, NC), lambda i: (i, 0))

_dense1 = pl.pallas_call(
    _dense1_body,
    grid=(N_NODES // BR,),
    in_specs=[_X_SPEC, _SP_SPEC, _DEG_SPEC, _W_SPEC, _W_SPEC, _B_SPEC],
    out_specs=_X_SPEC,
    out_shape=jax.ShapeDtypeStruct((N_NODES, D), jnp.float32),
)

_dense2 = pl.pallas_call(
    _dense2_body,
    grid=(N_NODES // BR,),
    in_specs=[_X_SPEC, _SP_SPEC, _DEG_SPEC, _W_SPEC, _W_SPEC, _B_SPEC,
              _W_SPEC, _B_SPEC],
    out_specs=_X_SPEC,
    out_shape=jax.ShapeDtypeStruct((N_NODES, D), jnp.float32),
)

_sc_agg_deg = _make_sc_agg(True)
_sc_agg = _make_sc_agg(False)


def kernel(features, edge_index, W_self1, W_neigh1, b1, W_self2, W_neigh2,
           b2, W_out, b_out):
    pad_n = NE_PAD - edge_index.shape[1]
    ar = jnp.arange(pad_n, dtype=jnp.int32)
    pad_src = (ar * 13) % N_NODES
    pad_dst = N_NODES + ar % (ACC_ROWS - N_NODES)
    src = jnp.concatenate([edge_index[0], pad_src])
    dst = jnp.concatenate([edge_index[1], pad_dst])
    z2d = jnp.zeros((ZPT, D), jnp.float32)
    z1d = jnp.zeros((ZPT,), jnp.float32)

    sp1, deg_flat = _sc_agg_deg(features, src, dst, z2d, z1d)
    degT = deg_flat.reshape(NC, N_NODES).T
    h1 = _dense1(features, sp1, degT, W_self1, W_neigh1, b1)
    sp2 = _sc_agg(h1, src, dst, z2d, z1d)

    Wo_p = jnp.zeros((D, D), jnp.float32).at[:, :N_CLASSES].set(W_out)
    bo_p = jnp.zeros((D,), jnp.float32).at[:N_CLASSES].set(b_out)
    out_p = _dense2(h1, sp2, degT, W_self2, W_neigh2, b2, Wo_p, bo_p)
    return out_p[:, :N_CLASSES]


# speedup vs baseline: 10.5077x; 1.7303x over previous
import functools

import jax
import jax.numpy as jnp
from jax import lax
from jax.experimental import pallas as pl
from jax.experimental.pallas import tpu as pltpu
from jax.experimental.pallas import tpu_sc as plsc

N_NODES = 10000
D = 128
N_CLASSES = 40

NC = 2
NS = 16
NW = NC * NS
K = 128
NE_PAD = 327680
EPW = NE_PAD // NW
NCH = EPW // K
NBUF = 4
ACC_ROWS = 10240
ZPT = ACC_ROWS // NS
RPT = 624


def _sc_agg_body(with_deg, table, src2d, dst2d, z2d, z1d, *rest):
    if with_deg:
        out, degout, acc, deg_s, sring, dbuf, rows, onesv, degb, gsem, isem = rest
    else:
        out, acc, sring, dbuf, rows, gsem, isem = rest
    c = lax.axis_index("c")
    s = lax.axis_index("s")
    w = c * NS + s
    ib = w * NCH

    pltpu.sync_copy(z2d, acc.at[pl.ds(s * ZPT, ZPT)])
    pltpu.sync_copy(dst2d.at[pl.ds(ib, NCH)], dbuf)
    for j in range(3):
        pltpu.sync_copy(src2d.at[ib + j], sring.at[j])
    if with_deg:
        pltpu.sync_copy(z1d, deg_s.at[pl.ds(s * ZPT, ZPT)])
        for j in range(K // 16):
            onesv[pl.ds(j * 16, 16)] = jnp.ones((16,), jnp.float32)
    plsc.subcore_barrier()

    pltpu.async_copy(src2d.at[ib + 3], sring.at[3], isem)
    pltpu.async_copy(table.at[sring.at[0]], rows.at[0], gsem)

    def step(i, carry):
        b = lax.rem(i, 2)
        pltpu.make_async_copy(table.at[sring.at[0]], rows.at[b], gsem).wait()

        @pl.when(i + 3 < NCH)
        def _():
            pltpu.make_async_copy(src2d.at[ib], sring.at[0], isem).wait()

        @pl.when(i + 1 < NCH)
        def _():
            pltpu.async_copy(table.at[sring.at[lax.rem(i + 1, 4)]],
                             rows.at[1 - b], gsem)

        @pl.when(i + 4 < NCH)
        def _():
            pltpu.async_copy(src2d.at[ib + i + 4],
                             sring.at[lax.rem(i, 4)], isem)

        pltpu.sync_copy(rows.at[b], acc.at[dbuf.at[i]], add=True)
        if with_deg:
            pltpu.sync_copy(onesv, deg_s.at[dbuf.at[i]], add=True)
        return carry

    lax.fori_loop(0, NCH, step, 0)
    plsc.subcore_barrier()

    pltpu.sync_copy(acc.at[pl.ds(s * RPT, RPT)], out.at[c, pl.ds(s * RPT, RPT)])
    if with_deg:
        pltpu.sync_copy(deg_s.at[pl.ds(s * RPT, RPT)], degb.at[pl.ds(0, RPT)])
        pltpu.sync_copy(degb.at[pl.ds(0, RPT)],
                        degout.at[pl.ds(c * N_NODES + s * RPT, RPT)])

    @pl.when(s == 0)
    def _():
        tail = N_NODES - NS * RPT
        pltpu.sync_copy(acc.at[pl.ds(NS * RPT, tail)],
                        out.at[c, pl.ds(NS * RPT, tail)])
        if with_deg:
            pltpu.sync_copy(deg_s.at[pl.ds(NS * RPT, tail)],
                            degb.at[pl.ds(0, tail)])
            pltpu.sync_copy(degb.at[pl.ds(0, tail)],
                            degout.at[pl.ds(c * N_NODES + NS * RPT, tail)])


def _make_sc_agg(with_deg):
    mesh = plsc.VectorSubcoreMesh(core_axis_name="c", subcore_axis_name="s")
    common = [
        pltpu.VMEM_SHARED((ACC_ROWS, D), jnp.float32),
    ]
    bufs = [
        pltpu.VMEM((4, K), jnp.int32),
        pltpu.VMEM((NCH, K), jnp.int32),
        pltpu.VMEM((2, K, D), jnp.float32),
    ]
    sems = [pltpu.SemaphoreType.DMA, pltpu.SemaphoreType.DMA]
    if with_deg:
        out_type = (
            jax.ShapeDtypeStruct((NC, N_NODES, D), jnp.float32),
            jax.ShapeDtypeStruct((NC * N_NODES,), jnp.float32),
        )
        scratch = common + [pltpu.VMEM_SHARED((ACC_ROWS,), jnp.float32)] \
            + bufs + [pltpu.VMEM((K,), jnp.float32),
                      pltpu.VMEM((ZPT,), jnp.float32)] + sems
    else:
        out_type = jax.ShapeDtypeStruct((NC, N_NODES, D), jnp.float32)
        scratch = common + bufs + sems
    return pl.kernel(
        functools.partial(_sc_agg_body, with_deg),
        out_type=out_type,
        mesh=mesh,
        scratch_types=scratch,
    )


BR = 1000


def _mean_agg(sp_ref, degT_ref):
    ssum = sp_ref[0] + sp_ref[1]
    deg = jnp.sum(degT_ref[...], axis=1, keepdims=True)
    invd = 1.0 / jnp.clip(deg, 1.0, None)
    return ssum * invd


def _dense1_body(x_ref, sp_ref, degT_ref, Ws_ref, Wn_ref, b_ref, o_ref):
    agg = _mean_agg(sp_ref, degT_ref)
    h = jnp.dot(x_ref[...], Ws_ref[...], preferred_element_type=jnp.float32)
    h = h + jnp.dot(agg, Wn_ref[...], preferred_element_type=jnp.float32)
    h = h + b_ref[...][None, :]
    o_ref[...] = jnp.maximum(h, 0.0)


def _dense2_body(x_ref, sp_ref, degT_ref, Ws_ref, Wn_ref, b_ref, Wo_ref,
                 bo_ref, o_ref):
    agg = _mean_agg(sp_ref, degT_ref)
    h = jnp.dot(x_ref[...], Ws_ref[...], preferred_element_type=jnp.float32)
    h = h + jnp.dot(agg, Wn_ref[...], preferred_element_type=jnp.float32)
    h = jnp.maximum(h + b_ref[...][None, :], 0.0)
    o_ref[...] = (
        jnp.dot(h, Wo_ref[...], preferred_element_type=jnp.float32)
        + bo_ref[...][None, :]
    )


_W_SPEC = pl.BlockSpec((D, D), lambda i: (0, 0))
_B_SPEC = pl.BlockSpec((D,), lambda i: (0,))
_X_SPEC = pl.BlockSpec((BR, D), lambda i: (i, 0))
_SP_SPEC = pl.BlockSpec((NC, BR, D), lambda i: (0, i, 0))
_DEG_SPEC = pl.BlockSpec((BR, NC), lambda i: (i, 0))

_dense1 = pl.pallas_call(
    _dense1_body,
    grid=(N_NODES // BR,),
    in_specs=[_X_SPEC, _SP_SPEC, _DEG_SPEC, _W_SPEC, _W_SPEC, _B_SPEC],
    out_specs=_X_SPEC,
    out_shape=jax.ShapeDtypeStruct((N_NODES, D), jnp.float32),
)

_dense2 = pl.pallas_call(
    _dense2_body,
    grid=(N_NODES // BR,),
    in_specs=[_X_SPEC, _SP_SPEC, _DEG_SPEC, _W_SPEC, _W_SPEC, _B_SPEC,
              _W_SPEC, _B_SPEC],
    out_specs=_X_SPEC,
    out_shape=jax.ShapeDtypeStruct((N_NODES, D), jnp.float32),
)

_sc_agg_deg = _make_sc_agg(True)
_sc_agg = _make_sc_agg(False)


def kernel(features, edge_index, W_self1, W_neigh1, b1, W_self2, W_neigh2,
           b2, W_out, b_out):
    pad_n = NE_PAD - edge_index.shape[1]
    ar = jnp.arange(pad_n, dtype=jnp.int32)
    pad_src = (ar * 13) % N_NODES
    pad_dst = N_NODES + ar % (ACC_ROWS - N_NODES)
    src2d = jnp.concatenate([edge_index[0], pad_src]).reshape(NW * NCH, K)
    dst2d = jnp.concatenate([edge_index[1], pad_dst]).reshape(NW * NCH, K)
    z2d = jnp.zeros((ZPT, D), jnp.float32)
    z1d = jnp.zeros((ZPT,), jnp.float32)

    sp1, deg_flat = _sc_agg_deg(features, src2d, dst2d, z2d, z1d)
    degT = deg_flat.reshape(NC, N_NODES).T
    h1 = _dense1(features, sp1, degT, W_self1, W_neigh1, b1)
    sp2 = _sc_agg(h1, src2d, dst2d, z2d, z1d)

    Wo_p = jnp.zeros((D, D), jnp.float32).at[:, :N_CLASSES].set(W_out)
    bo_p = jnp.zeros((D,), jnp.float32).at[:N_CLASSES].set(b_out)
    out_p = _dense2(h1, sp2, degT, W_self2, W_neigh2, b2, Wo_p, bo_p)
    return out_p[:, :N_CLASSES]
